# baseline (speedup 1.0000x reference)
import jax
import jax.numpy as jnp
from jax.experimental import pallas as pl
from jax.experimental.pallas import tpu as pltpu

_ROWS = 32 * 3 * 512
_COLS = 512
_BLOCK_ROWS = 2048
_GRID = _ROWS // _BLOCK_ROWS


_CHUNK = 8


def _masked_mse_block(inp_ref, tgt_ref, msk_ref, out_ref, acc_ref, cnt_ref):
    i = pl.program_id(0)

    accf = None
    acci = None
    for k in range(_BLOCK_ROWS // _CHUNK):
        lo = k * _CHUNK
        hi = lo + _CHUNK
        d = inp_ref[lo:hi, :] - tgt_ref[lo:hi, :]
        m = msk_ref[lo:hi, :]
        pf = d * d * m.astype(jnp.float32)
        accf = pf if accf is None else accf + pf
        acci = m if acci is None else acci + m

    @pl.when(i == 0)
    def _init():
        acc_ref[...] = accf
        cnt_ref[...] = acci

    @pl.when(i > 0)
    def _accum():
        acc_ref[...] += accf
        cnt_ref[...] += acci

    @pl.when(i == _GRID - 1)
    def _finalize():
        s = jnp.sum(acc_ref[...])
        c = jnp.sum(cnt_ref[...]).astype(jnp.float32)
        out_ref[0, 0] = s / c


def kernel(input, target, mask):
    x = input.reshape(_ROWS, _COLS)
    t = target.reshape(_ROWS, _COLS)
    mk = mask.reshape(_ROWS, _COLS)

    in_spec = pl.BlockSpec((_BLOCK_ROWS, _COLS), lambda i: (i, 0))

    res = pl.pallas_call(
        _masked_mse_block,
        grid=(_GRID,),
        in_specs=[in_spec, in_spec, in_spec],
        out_specs=pl.BlockSpec(memory_space=pltpu.SMEM),
        out_shape=jax.ShapeDtypeStruct((1, 1), jnp.float32),
        scratch_shapes=[
            pltpu.VMEM((_CHUNK, _COLS), jnp.float32),
            pltpu.VMEM((_CHUNK, _COLS), jnp.int32),
        ],
        compiler_params=pltpu.CompilerParams(
            dimension_semantics=("arbitrary",),
        ),
        name="masked_mse",
    )(x, t, mk)

    return res.reshape(())


# speedup vs baseline: 1.0021x; 1.0021x over previous
import jax
import jax.numpy as jnp
from jax.experimental import pallas as pl
from jax.experimental.pallas import tpu as pltpu

_ROWS = 32 * 3 * 512
_COLS = 512
_BLOCK_ROWS = 1024
_GRID = _ROWS // _BLOCK_ROWS


_CHUNK = 8


def _masked_mse_block(inp_ref, tgt_ref, msk_ref, out_ref, acc_ref, cnt_ref):
    i = pl.program_id(0)

    accf = None
    acci = None
    for k in range(_BLOCK_ROWS // _CHUNK):
        lo = k * _CHUNK
        hi = lo + _CHUNK
        d = inp_ref[lo:hi, :] - tgt_ref[lo:hi, :]
        m = msk_ref[lo:hi, :]
        pf = d * d * m.astype(jnp.float32)
        accf = pf if accf is None else accf + pf
        acci = m if acci is None else acci + m

    @pl.when(i == 0)
    def _init():
        acc_ref[...] = accf
        cnt_ref[...] = acci

    @pl.when(i > 0)
    def _accum():
        acc_ref[...] += accf
        cnt_ref[...] += acci

    @pl.when(i == _GRID - 1)
    def _finalize():
        s = jnp.sum(acc_ref[...])
        c = jnp.sum(cnt_ref[...]).astype(jnp.float32)
        out_ref[0, 0] = s / c


def kernel(input, target, mask):
    x = input.reshape(_ROWS, _COLS)
    t = target.reshape(_ROWS, _COLS)
    mk = mask.reshape(_ROWS, _COLS)

    in_spec = pl.BlockSpec((_BLOCK_ROWS, _COLS), lambda i: (i, 0))

    res = pl.pallas_call(
        _masked_mse_block,
        grid=(_GRID,),
        in_specs=[in_spec, in_spec, in_spec],
        out_specs=pl.BlockSpec(memory_space=pltpu.SMEM),
        out_shape=jax.ShapeDtypeStruct((1, 1), jnp.float32),
        scratch_shapes=[
            pltpu.VMEM((_CHUNK, _COLS), jnp.float32),
            pltpu.VMEM((_CHUNK, _COLS), jnp.int32),
        ],
        compiler_params=pltpu.CompilerParams(
            dimension_semantics=("arbitrary",),
        ),
        name="masked_mse",
    )(x, t, mk)

    return res.reshape(())
